# baseline (speedup 1.0000x reference)
import functools

import jax
import jax.numpy as jnp
from jax import lax
from jax.experimental import pallas as pl
from jax.experimental.pallas import tpu as pltpu
from jax.experimental.pallas import tpu_sc as plsc

_GAMMA = 12.0
_HIDDEN = 64
_BATCH = 16384
_NW = 32
_BPW = _BATCH // _NW
_CHUNK = 128
_NCHUNK = _BPW // _CHUNK
_GROUPS = _BPW // 16


def _sqrt16(x):
    xs = jnp.maximum(x, jnp.float32(1.1754944e-38))
    i = lax.bitcast_convert_type(xs, jnp.int32)
    i = jnp.int32(0x5F3759DF) - lax.shift_right_arithmetic(i, jnp.int32(1))
    y = lax.bitcast_convert_type(i, jnp.float32)
    for _ in range(3):
        y = y * (jnp.float32(1.5) - jnp.float32(0.5) * xs * y * y)
    return xs * y


def _sq_diff(h, t):
    ha, hb = plsc.unpack(h, format=plsc.PackFormat.INTERLEAVED)
    ta, tb = plsc.unpack(t, format=plsc.PackFormat.INTERLEAVED)
    da = ha - ta
    db = hb - tb
    return da * da + db * db


def _make_sc_kernel():
    mesh = plsc.VectorSubcoreMesh(core_axis_name="c", subcore_axis_name="s")

    @functools.partial(
        pl.kernel,
        mesh=mesh,
        compiler_params=pltpu.CompilerParams(
            needs_layout_passes=False, use_tc_tiling_on_sc=False),
        out_type=jax.ShapeDtypeStruct((_NW, _BPW), jnp.float32),
        scratch_types=[
            pltpu.VMEM((_NCHUNK, _CHUNK), jnp.int32),
            pltpu.VMEM((_NCHUNK, _CHUNK), jnp.int32),
            pltpu.VMEM((_BPW, _HIDDEN), jnp.bfloat16),
            pltpu.VMEM((_BPW, _HIDDEN), jnp.bfloat16),
            pltpu.VMEM((_BPW,), jnp.float32),
            pltpu.SemaphoreType.DMA,
        ],
    )
    def sc_kernel(table_hbm, hidx_hbm, tidx_hbm, out_hbm,
                  hidx_v, tidx_v, hrows_v, trows_v, scores_v, sem):
        wid = lax.axis_index("s") * 2 + lax.axis_index("c")

        pltpu.sync_copy(hidx_hbm.at[wid], hidx_v)
        pltpu.sync_copy(tidx_hbm.at[wid], tidx_v)

        copies = []
        for j in range(_NCHUNK):
            copies.append(pltpu.async_copy(
                table_hbm.at[hidx_v.at[j]],
                hrows_v.at[pl.ds(j * _CHUNK, _CHUNK)], sem))
            copies.append(pltpu.async_copy(
                table_hbm.at[tidx_v.at[j]],
                trows_v.at[pl.ds(j * _CHUNK, _CHUNK)], sem))
        for c in copies:
            c.wait()

        iota16 = lax.iota(jnp.int32, 16)

        def group(g, carry):
            sums = jnp.zeros((16,), jnp.float32)
            for j in range(16):
                r = g * 16 + j
                hrow = hrows_v.at[r]
                trow = trows_v.at[r]
                s0 = _sq_diff(hrow[pl.ds(0, 32)], trow[pl.ds(0, 32)])
                s1 = _sq_diff(hrow[pl.ds(32, 32)], trow[pl.ds(32, 32)])
                total = jnp.sum(s0 + s1)
                sums = jnp.where(iota16 == j, total, sums)
            scores_v[pl.ds(g * 16, 16)] = jnp.float32(_GAMMA) - _sqrt16(sums)
            return carry

        lax.fori_loop(0, _GROUPS, group, 0)

        pltpu.sync_copy(scores_v, out_hbm.at[wid])

    return sc_kernel


_sc_kernel = _make_sc_kernel()


@jax.jit
def kernel(sample, entity_embedding):
    table16 = entity_embedding.astype(jnp.bfloat16)
    hidx = sample[:, 0].reshape(_NW, _NCHUNK, _CHUNK)
    tidx = sample[:, 1].reshape(_NW, _NCHUNK, _CHUNK)
    out = _sc_kernel(table16, hidx, tidx)
    return out.reshape(_BATCH, 1)


# speedup vs baseline: 1.2896x; 1.2896x over previous
import functools

import jax
import jax.numpy as jnp
from jax import lax
from jax.experimental import pallas as pl
from jax.experimental.pallas import tpu as pltpu
from jax.experimental.pallas import tpu_sc as plsc

_GAMMA = 12.0
_HIDDEN = 64
_BATCH = 16384
_NW = 32
_BPW = _BATCH // _NW
_CHUNK = 128
_PAIRS = _CHUNK // 2
_NCHUNK = _BPW // _PAIRS


def _sqrt16(x):
    xs = jnp.maximum(x, jnp.float32(1.1754944e-38))
    i = lax.bitcast_convert_type(xs, jnp.int32)
    i = jnp.int32(0x5F3759DF) - lax.shift_right_arithmetic(i, jnp.int32(1))
    y = lax.bitcast_convert_type(i, jnp.float32)
    for _ in range(3):
        y = y * (jnp.float32(1.5) - jnp.float32(0.5) * xs * y * y)
    return xs * y


def _make_sc_kernel():
    mesh = plsc.VectorSubcoreMesh(core_axis_name="c", subcore_axis_name="s")

    @functools.partial(
        pl.kernel,
        mesh=mesh,
        compiler_params=pltpu.CompilerParams(
            needs_layout_passes=False, use_tc_tiling_on_sc=False),
        out_type=jax.ShapeDtypeStruct((_NW, _BPW), jnp.float32),
        scratch_types=[
            pltpu.VMEM((_NCHUNK, _CHUNK), jnp.int32),
            pltpu.VMEM((_NCHUNK, _CHUNK), jnp.int32),
            pltpu.VMEM((_CHUNK, 128), jnp.float32),
            pltpu.VMEM((_CHUNK, 128), jnp.float32),
            pltpu.VMEM((_BPW,), jnp.float32),
            pltpu.SemaphoreType.DMA,
        ],
    )
    def sc_kernel(table_hbm, idx_hbm, out_hbm, idx_v, rowid_v, rows0_v,
                  rows1_v, scores_v, sem):
        wid = lax.axis_index("s") * 2 + lax.axis_index("c")

        pltpu.sync_copy(idx_hbm.at[wid], idx_v)

        for c in range(_NCHUNK):
            for k in range(_CHUNK // 16):
                v = idx_v[c, pl.ds(k * 16, 16)]
                rowid_v[c, pl.ds(k * 16, 16)] = lax.shift_right_logical(
                    v, jnp.int32(1))

        def fire(c, buf):
            pltpu.async_copy(table_hbm.at[rowid_v.at[c]], buf, sem)

        def drain(buf):
            pltpu.make_async_copy(
                table_hbm.at[pl.ds(0, _CHUNK)], buf, sem).wait()

        iota16 = lax.iota(jnp.int32, 16)

        def compute(buf, c):
            cv = jnp.full((16,), 0, jnp.int32) + c

            def group(g, carry):
                pair = iota16 + g * 16
                hoff = (plsc.load_gather(idx_v, [cv, pair * 2])
                        & jnp.int32(1)) * jnp.int32(64)
                toff = (plsc.load_gather(idx_v, [cv, pair * 2 + 1])
                        & jnp.int32(1)) * jnp.int32(64)
                hrow = pair * 2
                trow = pair * 2 + 1
                accs = [jnp.zeros((16,), jnp.float32) for _ in range(4)]
                for d in range(_HIDDEN):
                    h = plsc.load_gather(buf, [hrow, hoff + d])
                    t = plsc.load_gather(buf, [trow, toff + d])
                    df = h - t
                    accs[d % 4] = accs[d % 4] + df * df
                total = (accs[0] + accs[1]) + (accs[2] + accs[3])
                scores_v[pl.ds(c * _PAIRS + g * 16, 16)] = (
                    jnp.float32(_GAMMA) - _sqrt16(total))
                return carry

            lax.fori_loop(0, _PAIRS // 16, group, 0)

        fire(0, rows0_v)
        fire(1, rows1_v)

        def pairbody(i, carry):
            c0 = 2 * i
            drain(rows0_v)
            compute(rows0_v, c0)
            fire(c0 + 2, rows0_v)
            drain(rows1_v)
            compute(rows1_v, c0 + 1)
            fire(c0 + 3, rows1_v)
            return carry

        lax.fori_loop(0, _NCHUNK // 2 - 1, pairbody, 0)

        drain(rows0_v)
        compute(rows0_v, _NCHUNK - 2)
        drain(rows1_v)
        compute(rows1_v, _NCHUNK - 1)

        pltpu.sync_copy(scores_v, out_hbm.at[wid])

    return sc_kernel


_sc_kernel = _make_sc_kernel()


@jax.jit
def kernel(sample, entity_embedding):
    table2 = entity_embedding.reshape(500000, 128)
    idx = sample.reshape(_NW, _NCHUNK, _CHUNK)
    out = _sc_kernel(table2, idx)
    return out.reshape(_BATCH, 1)


# speedup vs baseline: 1.2899x; 1.0002x over previous
import functools

import jax
import jax.numpy as jnp
from jax import lax
from jax.experimental import pallas as pl
from jax.experimental.pallas import tpu as pltpu
from jax.experimental.pallas import tpu_sc as plsc

_GAMMA = 12.0
_HIDDEN = 64
_BATCH = 16384
_NW = 32
_BPW = _BATCH // _NW
_CHUNK = 128
_PAIRS = _CHUNK // 2
_NCHUNK = _BPW // _PAIRS


def _sqrt16(x):
    xs = jnp.maximum(x, jnp.float32(1.1754944e-38))
    i = lax.bitcast_convert_type(xs, jnp.int32)
    i = jnp.int32(0x5F3759DF) - lax.shift_right_arithmetic(i, jnp.int32(1))
    y = lax.bitcast_convert_type(i, jnp.float32)
    for _ in range(3):
        y = y * (jnp.float32(1.5) - jnp.float32(0.5) * xs * y * y)
    return xs * y


def _make_sc_kernel():
    mesh = plsc.VectorSubcoreMesh(core_axis_name="c", subcore_axis_name="s")

    @functools.partial(
        pl.kernel,
        mesh=mesh,
        compiler_params=pltpu.CompilerParams(
            needs_layout_passes=False, use_tc_tiling_on_sc=True),
        out_type=jax.ShapeDtypeStruct((_NW, _BPW), jnp.float32),
        scratch_types=[
            pltpu.VMEM((_NCHUNK, _CHUNK), jnp.int32),
            pltpu.VMEM((_NCHUNK, _CHUNK), jnp.int32),
            pltpu.VMEM((_CHUNK, 128), jnp.float32),
            pltpu.VMEM((_CHUNK, 128), jnp.float32),
            pltpu.VMEM((_BPW,), jnp.float32),
            pltpu.SemaphoreType.DMA,
        ],
    )
    def sc_kernel(table_hbm, idx_hbm, out_hbm, idx_v, rowid_v, rows0_v,
                  rows1_v, scores_v, sem):
        wid = lax.axis_index("s") * 2 + lax.axis_index("c")

        pltpu.sync_copy(idx_hbm.at[wid], idx_v)

        for c in range(_NCHUNK):
            for k in range(_CHUNK // 16):
                v = idx_v[c, pl.ds(k * 16, 16)]
                rowid_v[c, pl.ds(k * 16, 16)] = lax.shift_right_logical(
                    v, jnp.int32(1))

        def fire(c, buf):
            pltpu.async_copy(table_hbm.at[rowid_v.at[c]], buf, sem)

        def drain(buf):
            pltpu.make_async_copy(
                table_hbm.at[pl.ds(0, _CHUNK)], buf, sem).wait()

        iota16 = lax.iota(jnp.int32, 16)

        def compute(buf, c):
            cv = jnp.full((16,), 0, jnp.int32) + c

            def group(g, carry):
                pair = iota16 + g * 16
                hoff = (plsc.load_gather(idx_v, [cv, pair * 2])
                        & jnp.int32(1)) * jnp.int32(64)
                toff = (plsc.load_gather(idx_v, [cv, pair * 2 + 1])
                        & jnp.int32(1)) * jnp.int32(64)
                hrow = pair * 2
                trow = pair * 2 + 1
                accs = [jnp.zeros((16,), jnp.float32) for _ in range(4)]
                for d in range(_HIDDEN):
                    h = plsc.load_gather(buf, [hrow, hoff + d])
                    t = plsc.load_gather(buf, [trow, toff + d])
                    df = h - t
                    accs[d % 4] = accs[d % 4] + df * df
                total = (accs[0] + accs[1]) + (accs[2] + accs[3])
                scores_v[pl.ds(c * _PAIRS + g * 16, 16)] = (
                    jnp.float32(_GAMMA) - _sqrt16(total))
                return carry

            lax.fori_loop(0, _PAIRS // 16, group, 0)

        fire(0, rows0_v)
        fire(1, rows1_v)

        def pairbody(i, carry):
            c0 = 2 * i
            drain(rows0_v)
            compute(rows0_v, c0)
            fire(c0 + 2, rows0_v)
            drain(rows1_v)
            compute(rows1_v, c0 + 1)
            fire(c0 + 3, rows1_v)
            return carry

        lax.fori_loop(0, _NCHUNK // 2 - 1, pairbody, 0)

        drain(rows0_v)
        compute(rows0_v, _NCHUNK - 2)
        drain(rows1_v)
        compute(rows1_v, _NCHUNK - 1)

        pltpu.sync_copy(scores_v, out_hbm.at[wid])

    return sc_kernel


_sc_kernel = _make_sc_kernel()


@jax.jit
def kernel(sample, entity_embedding):
    table2 = entity_embedding.reshape(500000, 128)
    idx = sample.reshape(_NW, _NCHUNK, _CHUNK)
    out = _sc_kernel(table2, idx)
    return out.reshape(_BATCH, 1)
